# Initial kernel scaffold; baseline (speedup 1.0000x reference)
import jax
import jax.numpy as jnp
from jax import lax
from jax.experimental import pallas as pl
from jax.experimental.pallas import tpu as pltpu
from jax.experimental.pallas import tpu_sc as plsc

N = 10000
E = 320000
D = 128

NC = 2
NS = 16
NW = NC * NS
EPT = E // NW
K = 80
NCHUNK = EPT // K
NPAD = 10240
NSLICE = NPAD // NS
ORPT = N // NS
OZROWS = 125

_mesh = plsc.VectorSubcoreMesh(core_axis_name="c", subcore_axis_name="s")


def _pre_body(h_ref, hq_ref, w1_ref, w2_ref, qs_ref, hs_ref):
    qs_ref[...] = jnp.sum(hq_ref[...] * w1_ref[...], axis=1, keepdims=True)
    hs_ref[...] = jnp.sum(h_ref[...] * w2_ref[...], axis=1, keepdims=True)


def _post_body(a_ref, b_ref, o_ref):
    o_ref[...] = a_ref[...] + b_ref[...]


def _sca_body(src_hbm, dst_hbm, ea0_hbm, ea1_hbm, qs_hbm, hs_hbm, par_hbm,
              ex_hbm, den_hbm, smax_hbm,
              srcc, dstc, ea0t, ea1t, qst, hst, part, at_, ext, mxt, mxall,
              zb, sden, smaxs):
    c = lax.axis_index("c")
    s = lax.axis_index("s")
    wid = c * NS + s
    rbase = wid * NCHUNK

    pltpu.sync_copy(src_hbm.at[pl.ds(rbase, NCHUNK)], srcc)
    pltpu.sync_copy(dst_hbm.at[pl.ds(rbase, NCHUNK)], dstc)
    pltpu.sync_copy(ea0_hbm.at[pl.ds(rbase, NCHUNK)], ea0t)
    pltpu.sync_copy(ea1_hbm.at[pl.ds(rbase, NCHUNK)], ea1t)
    pltpu.sync_copy(qs_hbm, qst)
    pltpu.sync_copy(hs_hbm, hst)
    pltpu.sync_copy(par_hbm, part)
    ew0 = part[0]
    ew1 = part[1]
    c0 = part[2]

    def _zb_step(i, _):
        zb[pl.ds(i * 16, 16)] = jnp.zeros((16,), jnp.float32)
        return 0
    lax.fori_loop(0, NSLICE // 16, _zb_step, 0)
    pltpu.sync_copy(zb, sden.at[pl.ds(s * NSLICE, NSLICE)])

    def _chunk_a(j, mx):
        for k in range(K // 16):
            sl = pl.ds(k * 16, 16)
            dstv = dstc[j, sl]
            srcv = srcc[j, sl]
            a16 = (plsc.load_gather(qst, [dstv])
                   + plsc.load_gather(hst, [srcv])
                   + ea0t[j, sl] * ew0 + ea1t[j, sl] * ew1 + c0)
            at_[j, sl] = a16
            mx = jnp.maximum(mx, a16)
        return mx
    mx = lax.fori_loop(0, NCHUNK, _chunk_a,
                       jnp.full((16,), -jnp.inf, jnp.float32))
    mxt[...] = mx
    pltpu.sync_copy(mxt, smaxs.at[s])

    plsc.subcore_barrier()

    pltpu.sync_copy(smaxs, mxall)
    mm = mxall[0]
    for s2 in range(1, NS):
        mm = jnp.maximum(mm, mxall[s2])
    m_s = jnp.max(mm)

    def _chunk_b(j, _):
        for k in range(K // 16):
            sl = pl.ds(k * 16, 16)
            ext[j, sl] = jnp.exp(at_[j, sl] - m_s)
        pltpu.sync_copy(ext.at[j], sden.at[dstc.at[j]], add=True)
        return 0
    lax.fori_loop(0, NCHUNK, _chunk_b, 0)
    pltpu.sync_copy(ext, ex_hbm.at[pl.ds(rbase, NCHUNK)])

    @pl.when(s == 0)
    def _():
        mxt[...] = jnp.full((16,), m_s, jnp.float32)
        pltpu.sync_copy(mxt, smax_hbm.at[pl.ds(c * 16, 16)])

    plsc.subcore_barrier()
    pltpu.sync_copy(sden.at[pl.ds(s * NSLICE, NSLICE)],
                    den_hbm.at[c, pl.ds(s * NSLICE, NSLICE)])


def _scb_body(src_hbm, dst_hbm, ex_hbm, den_hbm, smax_hbm, h_hbm,
              outp_hbm,
              srcc, dstc, exc, dent, dtmp, smaxt, ab, rows, zb, gsem, sout):
    c = lax.axis_index("c")
    s = lax.axis_index("s")
    wid = c * NS + s
    rbase = wid * NCHUNK

    pltpu.sync_copy(src_hbm.at[pl.ds(rbase, NCHUNK)], srcc)
    pltpu.sync_copy(dst_hbm.at[pl.ds(rbase, NCHUNK)], dstc)
    pltpu.sync_copy(ex_hbm.at[pl.ds(rbase, NCHUNK)], exc)
    pltpu.sync_copy(den_hbm.at[0], dent)
    pltpu.sync_copy(den_hbm.at[1], dtmp)
    pltpu.sync_copy(smax_hbm, smaxt)

    m0 = smaxt[pl.ds(0, 16)]
    m1 = smaxt[pl.ds(16, 16)]
    mg = jnp.maximum(m0, m1)
    s0 = jnp.exp(m0 - mg)
    s1 = jnp.exp(m1 - mg)
    myv = jnp.where(c == 0, s0, s1)

    def _den_step(i, _):
        sl = pl.ds(i * 16, 16)
        dent[sl] = dent[sl] * s0 + dtmp[sl] * s1
        return 0
    lax.fori_loop(0, NPAD // 16, _den_step, 0)

    def _zb_step(i, _):
        r = i // (D // 16)
        u = i % (D // 16)
        zb[r, pl.ds(u * 16, 16)] = jnp.zeros((16,), jnp.float32)
        return 0
    lax.fori_loop(0, OZROWS * (D // 16), _zb_step, 0)
    for t in range(ORPT // OZROWS):
        pltpu.sync_copy(zb, sout.at[pl.ds(s * ORPT + t * OZROWS, OZROWS)])

    plsc.subcore_barrier()

    def _chunk(j, _):
        pltpu.async_copy(h_hbm.at[srcc.at[j]], rows, gsem).wait()
        for k in range(K // 16):
            sl = pl.ds(k * 16, 16)
            dstv = dstc[j, sl]
            denv = plsc.load_gather(dent, [dstv])
            ab[sl] = exc[j, sl] * myv / (denv + 1e-16)
        def _row(r, _2):
            al = ab[r]
            for u in range(D // 16):
                su = pl.ds(u * 16, 16)
                rows[r, su] = rows[r, su] * al
            return 0
        lax.fori_loop(0, K, _row, 0)
        pltpu.sync_copy(rows, sout.at[dstc.at[j]], add=True)
        return 0
    lax.fori_loop(0, NCHUNK, _chunk, 0)

    plsc.subcore_barrier()
    pltpu.sync_copy(sout.at[pl.ds(s * ORPT, ORPT)],
                    outp_hbm.at[c, pl.ds(s * ORPT, ORPT)])


_pre_call = pl.pallas_call(
    _pre_body,
    out_shape=(jax.ShapeDtypeStruct((N, 1), jnp.float32),
               jax.ShapeDtypeStruct((N, 1), jnp.float32)),
)

_post_call = pl.pallas_call(
    _post_body,
    out_shape=jax.ShapeDtypeStruct((N, D), jnp.float32),
)

_ka_call = pl.kernel(
    _sca_body,
    out_type=(jax.ShapeDtypeStruct((NW * NCHUNK, K), jnp.float32),
              jax.ShapeDtypeStruct((NC, NPAD), jnp.float32),
              jax.ShapeDtypeStruct((NC * 16,), jnp.float32)),
    mesh=_mesh,
    scratch_types=[
        pltpu.VMEM((NCHUNK, K), jnp.int32),
        pltpu.VMEM((NCHUNK, K), jnp.int32),
        pltpu.VMEM((NCHUNK, K), jnp.float32),
        pltpu.VMEM((NCHUNK, K), jnp.float32),
        pltpu.VMEM((N,), jnp.float32),
        pltpu.VMEM((N,), jnp.float32),
        pltpu.VMEM((16,), jnp.float32),
        pltpu.VMEM((NCHUNK, K), jnp.float32),
        pltpu.VMEM((NCHUNK, K), jnp.float32),
        pltpu.VMEM((16,), jnp.float32),
        pltpu.VMEM((NS, 16), jnp.float32),
        pltpu.VMEM((NSLICE,), jnp.float32),
        pltpu.VMEM_SHARED((NPAD,), jnp.float32),
        pltpu.VMEM_SHARED((NS, 16), jnp.float32),
    ],
)

_kb_call = pl.kernel(
    _scb_body,
    out_type=jax.ShapeDtypeStruct((NC, N, D), jnp.float32),
    mesh=_mesh,
    scratch_types=[
        pltpu.VMEM((NCHUNK, K), jnp.int32),
        pltpu.VMEM((NCHUNK, K), jnp.int32),
        pltpu.VMEM((NCHUNK, K), jnp.float32),
        pltpu.VMEM((NPAD,), jnp.float32),
        pltpu.VMEM((NPAD,), jnp.float32),
        pltpu.VMEM((NC * 16,), jnp.float32),
        pltpu.VMEM((K,), jnp.float32),
        pltpu.VMEM((K, D), jnp.float32),
        pltpu.VMEM((OZROWS, D), jnp.float32),
        pltpu.SemaphoreType.DMA,
        pltpu.VMEM_SHARED((N, D), jnp.float32),
    ],
)


def kernel(h, edge_index, h_attn_q, edge_attr, We, be, Wa, ba):
    src = edge_index[0].reshape(NW * NCHUNK, K)
    dst = edge_index[1].reshape(NW * NCHUNK, K)
    ea0 = edge_attr[:, 0].reshape(NW * NCHUNK, K)
    ea1 = edge_attr[:, 1].reshape(NW * NCHUNK, K)
    w1 = Wa[:D, 0]
    w2 = Wa[D:, 0]
    ew = We @ w2
    c0 = be @ w2 + ba[0]
    par = jnp.zeros((16,), jnp.float32).at[0].set(ew[0]).at[1].set(ew[1]) \
        .at[2].set(c0)

    qs2, hs2 = _pre_call(h, h_attn_q, w1.reshape(1, D), w2.reshape(1, D))
    qs = qs2.reshape(N)
    hs = hs2.reshape(N)

    ex, den, smax = _ka_call(src, dst, ea0, ea1, qs, hs, par)
    outp = _kb_call(src, dst, ex, den, smax, h)
    return _post_call(outp[0], outp[1])


# speedup vs baseline: 10.6955x; 10.6955x over previous
import functools

import jax
import jax.numpy as jnp
from jax import lax
from jax.experimental import pallas as pl
from jax.experimental.pallas import tpu as pltpu
from jax.experimental.pallas import tpu_sc as plsc

N = 10000
E = 320000
D = 128

NC = 2
NS = 16
NW = NC * NS
EPT = E // NW
K = 80
NCHUNK = EPT // K
NPAD = 10240
NSLICE = NPAD // NS
NPADR = 10240
ORPT = NPADR // NS
OZROWS = 128


def _pre_body(h_ref, hq_ref, w1_ref, w2_ref, qs_ref, hs_ref):
    qs_ref[...] = jnp.sum(hq_ref[...] * w1_ref[...], axis=1, keepdims=True)
    hs_ref[...] = jnp.sum(h_ref[...] * w2_ref[...], axis=1, keepdims=True)


def _post_body(a_ref, b_ref, o_ref):
    o_ref[...] = jnp.concatenate(
        [a_ref[pl.ds(0, N), :], b_ref[pl.ds(0, N), :]], axis=1)


def _sca_body(srcf_hbm, dst3_hbm, ea0_hbm, ea1_hbm, qs_hbm, hs_hbm,
              par_hbm,
              ex_hbm, den_hbm, smax_hbm,
              srcf, dstc, ea0t, ea1t, qst, hst, part, at_, ext, mxt,
              mxall, zb, sden, smaxs):
    c = lax.axis_index("c")
    s = lax.axis_index("s")
    wid = c * NS + s
    ebase = wid * EPT

    pltpu.sync_copy(srcf_hbm.at[pl.ds(ebase, EPT)], srcf)
    pltpu.sync_copy(dst3_hbm.at[wid], dstc)
    pltpu.sync_copy(ea0_hbm.at[pl.ds(ebase, EPT)], ea0t)
    pltpu.sync_copy(ea1_hbm.at[pl.ds(ebase, EPT)], ea1t)
    pltpu.sync_copy(qs_hbm, qst)
    pltpu.sync_copy(hs_hbm, hst)
    pltpu.sync_copy(par_hbm, part)
    pv = part[...]
    ew0 = pv[0]
    ew1 = pv[1]
    c0 = pv[2]

    def _zb_step(i, _):
        zb[pl.ds(i * 16, 16)] = jnp.zeros((16,), jnp.float32)
        return 0
    lax.fori_loop(0, NSLICE // 16, _zb_step, 0)
    pltpu.sync_copy(zb, sden.at[pl.ds(s * NSLICE, NSLICE)])

    def _step_a(j, mx):
        for k in range(K // 16):
            sl = pl.ds(j * K + k * 16, 16)
            a16 = (plsc.load_gather(qst, [dstc[j, pl.ds(k * 16, 16)]])
                   + plsc.load_gather(hst, [srcf[sl]])
                   + ea0t[sl] * ew0 + ea1t[sl] * ew1 + c0)
            at_[sl] = a16
            mx = jnp.maximum(mx, a16)
        return mx
    mx = lax.fori_loop(0, NCHUNK, _step_a,
                       jnp.full((16,), -jnp.inf, jnp.float32))
    mxt[...] = mx
    pltpu.sync_copy(mxt, smaxs.at[s])

    plsc.subcore_barrier()

    pltpu.sync_copy(smaxs, mxall)
    mm = mxall[0]
    for s2 in range(1, NS):
        mm = jnp.maximum(mm, mxall[s2])
    m_s = jnp.max(mm)

    def _chunk_b(j, _):
        for k in range(K // 16):
            sl = pl.ds(j * K + k * 16, 16)
            ext[sl] = jnp.exp(at_[sl] - m_s)
        pltpu.sync_copy(ext.at[pl.ds(j * K, K)], sden.at[dstc.at[j]],
                        add=True)
        return 0
    lax.fori_loop(0, NCHUNK, _chunk_b, 0)
    pltpu.sync_copy(ext, ex_hbm.at[pl.ds(ebase, EPT)])

    @pl.when(s == 0)
    def _():
        mxt[...] = jnp.full((16,), m_s, jnp.float32)
        pltpu.sync_copy(mxt, smax_hbm.at[pl.ds(c * 16, 16)])

    plsc.subcore_barrier()
    pltpu.sync_copy(sden.at[pl.ds(s * NSLICE, NSLICE)],
                    den_hbm.at[pl.ds(c * NPAD + s * NSLICE, NSLICE)])


DH = D // NC
EPT2 = E // NS
NCHUNK2 = EPT2 // K


def _scb_body(src3_hbm, dst3_hbm, ex_hbm, den_hbm, smax_hbm, hsplit_hbm,
              outp_hbm,
              srcc, dstc, exc, dent, d0s, d1s, smaxt, ab, rows, zb, gsem,
              sdc, sout):
    c = lax.axis_index("c")
    s = lax.axis_index("s")
    ebase = s * EPT2

    pltpu.sync_copy(src3_hbm.at[s], srcc)
    pltpu.sync_copy(dst3_hbm.at[s], dstc)
    pltpu.sync_copy(ex_hbm.at[pl.ds(ebase, EPT2)], exc)
    pltpu.sync_copy(den_hbm.at[pl.ds(s * NSLICE, NSLICE)], d0s)
    pltpu.sync_copy(den_hbm.at[pl.ds(NPAD + s * NSLICE, NSLICE)], d1s)
    pltpu.sync_copy(smax_hbm, smaxt)

    m0 = smaxt[pl.ds(0, 16)]
    m1 = smaxt[pl.ds(16, 16)]
    mg = jnp.maximum(m0, m1)
    s0 = jnp.exp(m0 - mg)
    s1 = jnp.exp(m1 - mg)
    myv = jnp.where(s < NS // 2, s0, s1)

    def _den_step(i, _):
        sl = pl.ds(i * 16, 16)
        d0s[sl] = d0s[sl] * s0 + d1s[sl] * s1
        return 0
    lax.fori_loop(0, NSLICE // 16, _den_step, 0)
    pltpu.sync_copy(d0s, sdc.at[pl.ds(s * NSLICE, NSLICE)])

    def _zb_step(i, _):
        r = i // (DH // 16)
        u = i % (DH // 16)
        zb[r, pl.ds(u * 16, 16)] = jnp.zeros((16,), jnp.float32)
        return 0
    lax.fori_loop(0, OZROWS * (DH // 16), _zb_step, 0)
    for t in range(ORPT // OZROWS):
        pltpu.sync_copy(zb, sout.at[pl.ds(s * ORPT + t * OZROWS, OZROWS)])

    plsc.subcore_barrier()
    pltpu.sync_copy(sdc, dent)

    def _chunk(j, _):
        pltpu.async_copy(hsplit_hbm.at[c].at[srcc.at[j]], rows, gsem).wait()
        for k in range(K // 16):
            sl = pl.ds(k * 16, 16)
            dstv = dstc[j, sl]
            denv = plsc.load_gather(dent, [dstv])
            ab[sl] = exc[pl.ds(j * K + k * 16, 16)] * myv / (denv + 1e-16)
        def _rowgrp(g, _2):
            al16 = ab[pl.ds(g * 16, 16)]
            for i in range(16):
                r = g * 16 + i
                al = al16[i]
                for u in range(DH // 16):
                    su = pl.ds(u * 16, 16)
                    rows[r, su] = rows[r, su] * al
            return 0
        lax.fori_loop(0, K // 16, _rowgrp, 0)
        pltpu.sync_copy(rows, sout.at[dstc.at[j]], add=True)
        return 0
    lax.fori_loop(0, NCHUNK2, _chunk, 0)

    plsc.subcore_barrier()
    pltpu.sync_copy(sout.at[pl.ds(s * ORPT, ORPT)],
                    outp_hbm.at[c, pl.ds(s * ORPT, ORPT)])


_pre_call = pl.pallas_call(
    _pre_body,
    out_shape=(jax.ShapeDtypeStruct((N, 1), jnp.float32),
               jax.ShapeDtypeStruct((N, 1), jnp.float32)),
)

_post_call = pl.pallas_call(
    _post_body,
    out_shape=jax.ShapeDtypeStruct((N, D), jnp.float32),
)


@functools.cache
def _sc_calls():
  mesh = plsc.VectorSubcoreMesh(core_axis_name="c", subcore_axis_name="s",
                                num_cores=NC, num_subcores=NS)
  cp = pltpu.CompilerParams(needs_layout_passes=False,
                            use_tc_tiling_on_sc=False)
  ka_call = pl.kernel(
    _sca_body,
    compiler_params=cp,
    out_type=(jax.ShapeDtypeStruct((E,), jnp.float32),
              jax.ShapeDtypeStruct((NC * NPAD,), jnp.float32),
              jax.ShapeDtypeStruct((NC * 16,), jnp.float32)),
    mesh=mesh,
    scratch_types=[
        pltpu.VMEM((EPT,), jnp.int32),
        pltpu.VMEM((NCHUNK, K), jnp.int32),
        pltpu.VMEM((EPT,), jnp.float32),
        pltpu.VMEM((EPT,), jnp.float32),
        pltpu.VMEM((N,), jnp.float32),
        pltpu.VMEM((N,), jnp.float32),
        pltpu.VMEM((16,), jnp.float32),
        pltpu.VMEM((EPT,), jnp.float32),
        pltpu.VMEM((EPT,), jnp.float32),
        pltpu.VMEM((16,), jnp.float32),
        pltpu.VMEM((NS, 16), jnp.float32),
        pltpu.VMEM((NSLICE,), jnp.float32),
        pltpu.VMEM_SHARED((NPAD,), jnp.float32),
        pltpu.VMEM_SHARED((NS, 16), jnp.float32),
    ],
  )
  kb_call = pl.kernel(
    _scb_body,
    compiler_params=cp,
    out_type=jax.ShapeDtypeStruct((NC, NPADR, DH), jnp.float32),
    mesh=mesh,
    scratch_types=[
        pltpu.VMEM((NCHUNK2, K), jnp.int32),
        pltpu.VMEM((NCHUNK2, K), jnp.int32),
        pltpu.VMEM((EPT2,), jnp.float32),
        pltpu.VMEM((NPAD,), jnp.float32),
        pltpu.VMEM((NSLICE,), jnp.float32),
        pltpu.VMEM((NSLICE,), jnp.float32),
        pltpu.VMEM((NC * 16,), jnp.float32),
        pltpu.VMEM((K,), jnp.float32),
        pltpu.VMEM((K, DH), jnp.float32),
        pltpu.VMEM((OZROWS, DH), jnp.float32),
        pltpu.SemaphoreType.DMA,
        pltpu.VMEM_SHARED((NPAD,), jnp.float32),
        pltpu.VMEM_SHARED((NPADR, DH), jnp.float32),
    ],
  )
  return ka_call, kb_call


def kernel(h, edge_index, h_attn_q, edge_attr, We, be, Wa, ba):
    srcf = edge_index[0]
    dstf = edge_index[1]
    src3 = srcf.reshape(NS, NCHUNK2, K)
    dst3 = dstf.reshape(NS, NCHUNK2, K)
    hsplit = h.reshape(N, NC, DH).transpose(1, 0, 2)
    ea0 = edge_attr[:, 0]
    ea1 = edge_attr[:, 1]
    w1 = Wa[:D, 0]
    w2 = Wa[D:, 0]
    ew = We @ w2
    c0 = be @ w2 + ba[0]
    par = jnp.zeros((16,), jnp.float32).at[0].set(ew[0]).at[1].set(ew[1]) \
        .at[2].set(c0)

    qs2, hs2 = _pre_call(h, h_attn_q, w1.reshape(1, D), w2.reshape(1, D))
    qs = qs2.reshape(N)
    hs = hs2.reshape(N)

    ka_call, kb_call = _sc_calls()
    dst3a = dstf.reshape(NW, NCHUNK, K)
    ex, den, smax = ka_call(srcf, dst3a, ea0, ea1, qs, hs, par)
    outp = kb_call(src3, dst3, ex, den, smax, hsplit)
    return _post_call(outp[0], outp[1])


# speedup vs baseline: 14.5488x; 1.3603x over previous
import functools

import jax
import jax.numpy as jnp
from jax import lax
from jax.experimental import pallas as pl
from jax.experimental.pallas import tpu as pltpu
from jax.experimental.pallas import tpu_sc as plsc

N = 10000
E = 320000
D = 128

NC = 2
NS = 16
NW = NC * NS
EPT = E // NW
K = 80
NCHUNK = EPT // K
NPAD = 10240
NSLICE = NPAD // NS
NPADR = 10240
ORPT = NPADR // NS
OZROWS = 64


def _pre_body(h_ref, hq_ref, w1_ref, w2_ref, qs_ref, hs_ref):
    qs_ref[...] = jnp.sum(hq_ref[...] * w1_ref[...], axis=1, keepdims=True)
    hs_ref[...] = jnp.sum(h_ref[...] * w2_ref[...], axis=1, keepdims=True)


def _post_body(a_ref, b_ref, o_ref):
    o_ref[...] = jnp.concatenate(
        [a_ref[pl.ds(0, N), :], b_ref[pl.ds(0, N), :]], axis=1)


def _sca_body(srcf_hbm, dst3_hbm, ea0_hbm, ea1_hbm, qs_hbm, hs_hbm,
              par_hbm,
              ex_hbm, den_hbm, smax_hbm,
              srcf, dstc, ea0t, ea1t, qst, hst, part, at_, ext, mxt,
              mxall, zb, sden, smaxs):
    c = lax.axis_index("c")
    s = lax.axis_index("s")
    wid = c * NS + s
    ebase = wid * EPT

    pltpu.sync_copy(srcf_hbm.at[pl.ds(ebase, EPT)], srcf)
    pltpu.sync_copy(dst3_hbm.at[wid], dstc)
    pltpu.sync_copy(ea0_hbm.at[pl.ds(ebase, EPT)], ea0t)
    pltpu.sync_copy(ea1_hbm.at[pl.ds(ebase, EPT)], ea1t)
    pltpu.sync_copy(qs_hbm, qst)
    pltpu.sync_copy(hs_hbm, hst)
    pltpu.sync_copy(par_hbm, part)
    pv = part[...]
    ew0 = pv[0]
    ew1 = pv[1]
    c0 = pv[2]

    def _zb_step(i, _):
        zb[pl.ds(i * 16, 16)] = jnp.zeros((16,), jnp.float32)
        return 0
    lax.fori_loop(0, NSLICE // 16, _zb_step, 0)
    pltpu.sync_copy(zb, sden.at[pl.ds(s * NSLICE, NSLICE)])

    def _step_a(j, mx):
        for k in range(K // 16):
            sl = pl.ds(j * K + k * 16, 16)
            a16 = (plsc.load_gather(qst, [dstc[j, pl.ds(k * 16, 16)]])
                   + plsc.load_gather(hst, [srcf[sl]])
                   + ea0t[sl] * ew0 + ea1t[sl] * ew1 + c0)
            at_[sl] = a16
            mx = jnp.maximum(mx, a16)
        return mx
    mx = lax.fori_loop(0, NCHUNK, _step_a,
                       jnp.full((16,), -jnp.inf, jnp.float32))
    mxt[...] = mx
    pltpu.sync_copy(mxt, smaxs.at[s])

    plsc.subcore_barrier()

    pltpu.sync_copy(smaxs, mxall)
    mm = mxall[0]
    for s2 in range(1, NS):
        mm = jnp.maximum(mm, mxall[s2])
    m_s = jnp.max(mm)

    def _chunk_b(j, _):
        for k in range(K // 16):
            sl = pl.ds(j * K + k * 16, 16)
            ext[sl] = jnp.exp(at_[sl] - m_s)
        pltpu.sync_copy(ext.at[pl.ds(j * K, K)], sden.at[dstc.at[j]],
                        add=True)
        return 0
    lax.fori_loop(0, NCHUNK, _chunk_b, 0)
    pltpu.sync_copy(ext, ex_hbm.at[pl.ds(ebase, EPT)])

    @pl.when(s == 0)
    def _():
        mxt[...] = jnp.full((16,), m_s, jnp.float32)
        pltpu.sync_copy(mxt, smax_hbm.at[pl.ds(c * 16, 16)])

    plsc.subcore_barrier()
    pltpu.sync_copy(sden.at[pl.ds(s * NSLICE, NSLICE)],
                    den_hbm.at[pl.ds(c * NPAD + s * NSLICE, NSLICE)])


DH = D // NC
EPT2 = E // NS
NCHUNK2 = EPT2 // K


def _scb_body(src3_hbm, dst3_hbm, ex_hbm, den_hbm, smax_hbm, hsplit_hbm,
              outp_hbm,
              srcc, dstc, exc, dent, d0s, d1s, smaxt, ab, rows0, rows1, zb,
              gsem0, gsem1, ssem0, ssem1, sdc, sout):
    c = lax.axis_index("c")
    s = lax.axis_index("s")
    ebase = s * EPT2

    pltpu.sync_copy(src3_hbm.at[s], srcc)
    pltpu.sync_copy(dst3_hbm.at[s], dstc)
    pltpu.sync_copy(ex_hbm.at[pl.ds(ebase, EPT2)], exc)
    pltpu.sync_copy(den_hbm.at[pl.ds(s * NSLICE, NSLICE)], d0s)
    pltpu.sync_copy(den_hbm.at[pl.ds(NPAD + s * NSLICE, NSLICE)], d1s)
    pltpu.sync_copy(smax_hbm, smaxt)

    m0 = smaxt[pl.ds(0, 16)]
    m1 = smaxt[pl.ds(16, 16)]
    mg = jnp.maximum(m0, m1)
    s0 = jnp.exp(m0 - mg)
    s1 = jnp.exp(m1 - mg)
    myv = jnp.where(s < NS // 2, s0, s1)

    def _den_step(i, _):
        sl = pl.ds(i * 16, 16)
        d0s[sl] = d0s[sl] * s0 + d1s[sl] * s1
        return 0
    lax.fori_loop(0, NSLICE // 16, _den_step, 0)
    pltpu.sync_copy(d0s, sdc.at[pl.ds(s * NSLICE, NSLICE)])

    def _zb_step(i, _):
        r = i // (DH // 16)
        u = i % (DH // 16)
        zb[r, pl.ds(u * 16, 16)] = jnp.zeros((16,), jnp.float32)
        return 0
    lax.fori_loop(0, OZROWS * (DH // 16), _zb_step, 0)
    for t in range(ORPT // OZROWS):
        pltpu.sync_copy(zb, sout.at[pl.ds(s * ORPT + t * OZROWS, OZROWS)])

    plsc.subcore_barrier()
    pltpu.sync_copy(sdc, dent)

    rowsb = (rows0, rows1)
    gsems = (gsem0, gsem1)
    ssems = (ssem0, ssem1)

    pltpu.async_copy(hsplit_hbm.at[c].at[srcc.at[0]], rows0, gsem0)

    def _pair(pp, _):
        for b in range(2):
            j = pp * 2 + b
            bn = 1 - b
            jn = j + 1

            @pl.when(j >= 1)
            def _():
                pltpu.make_async_copy(rowsb[bn], sout.at[dstc.at[j]],
                                      ssems[bn]).wait()

            @pl.when(jn < NCHUNK2)
            def _():
                pltpu.async_copy(hsplit_hbm.at[c].at[srcc.at[jn]],
                                 rowsb[bn], gsems[bn])

            pltpu.make_async_copy(hsplit_hbm.at[c].at[srcc.at[j]],
                                  rowsb[b], gsems[b]).wait()
            for k in range(K // 16):
                sl = pl.ds(k * 16, 16)
                dstv = dstc[j, sl]
                denv = plsc.load_gather(dent, [dstv])
                ab[sl] = exc[pl.ds(j * K + k * 16, 16)] * myv / (denv + 1e-16)

            def _rowgrp(g, _2):
                al16 = ab[pl.ds(g * 16, 16)]
                for i in range(16):
                    r = g * 16 + i
                    al = al16[i]
                    for u in range(DH // 16):
                        su = pl.ds(u * 16, 16)
                        rowsb[b][r, su] = rowsb[b][r, su] * al
                return 0
            lax.fori_loop(0, K // 16, _rowgrp, 0)
            pltpu.async_copy(rowsb[b], sout.at[dstc.at[j]], ssems[b],
                             add=True)
        return 0
    lax.fori_loop(0, NCHUNK2 // 2, _pair, 0)
    pltpu.make_async_copy(rows1, sout.at[dstc.at[0]], ssem1).wait()

    plsc.subcore_barrier()
    pltpu.sync_copy(sout.at[pl.ds(s * ORPT, ORPT)],
                    outp_hbm.at[c, pl.ds(s * ORPT, ORPT)])


_pre_call = pl.pallas_call(
    _pre_body,
    out_shape=(jax.ShapeDtypeStruct((N, 1), jnp.float32),
               jax.ShapeDtypeStruct((N, 1), jnp.float32)),
)

_post_call = pl.pallas_call(
    _post_body,
    out_shape=jax.ShapeDtypeStruct((N, D), jnp.float32),
)


@functools.cache
def _sc_calls():
  mesh = plsc.VectorSubcoreMesh(core_axis_name="c", subcore_axis_name="s",
                                num_cores=NC, num_subcores=NS)
  cp = pltpu.CompilerParams(needs_layout_passes=False,
                            use_tc_tiling_on_sc=False)
  ka_call = pl.kernel(
    _sca_body,
    compiler_params=cp,
    out_type=(jax.ShapeDtypeStruct((E,), jnp.float32),
              jax.ShapeDtypeStruct((NC * NPAD,), jnp.float32),
              jax.ShapeDtypeStruct((NC * 16,), jnp.float32)),
    mesh=mesh,
    scratch_types=[
        pltpu.VMEM((EPT,), jnp.int32),
        pltpu.VMEM((NCHUNK, K), jnp.int32),
        pltpu.VMEM((EPT,), jnp.float32),
        pltpu.VMEM((EPT,), jnp.float32),
        pltpu.VMEM((N,), jnp.float32),
        pltpu.VMEM((N,), jnp.float32),
        pltpu.VMEM((16,), jnp.float32),
        pltpu.VMEM((EPT,), jnp.float32),
        pltpu.VMEM((EPT,), jnp.float32),
        pltpu.VMEM((16,), jnp.float32),
        pltpu.VMEM((NS, 16), jnp.float32),
        pltpu.VMEM((NSLICE,), jnp.float32),
        pltpu.VMEM_SHARED((NPAD,), jnp.float32),
        pltpu.VMEM_SHARED((NS, 16), jnp.float32),
    ],
  )
  kb_call = pl.kernel(
    _scb_body,
    compiler_params=cp,
    out_type=jax.ShapeDtypeStruct((NC, NPADR, DH), jnp.float32),
    mesh=mesh,
    scratch_types=[
        pltpu.VMEM((NCHUNK2, K), jnp.int32),
        pltpu.VMEM((NCHUNK2, K), jnp.int32),
        pltpu.VMEM((EPT2,), jnp.float32),
        pltpu.VMEM((NPAD,), jnp.float32),
        pltpu.VMEM((NSLICE,), jnp.float32),
        pltpu.VMEM((NSLICE,), jnp.float32),
        pltpu.VMEM((NC * 16,), jnp.float32),
        pltpu.VMEM((K,), jnp.float32),
        pltpu.VMEM((K, DH), jnp.float32),
        pltpu.VMEM((K, DH), jnp.float32),
        pltpu.VMEM((OZROWS, DH), jnp.float32),
        pltpu.SemaphoreType.DMA,
        pltpu.SemaphoreType.DMA,
        pltpu.SemaphoreType.DMA,
        pltpu.SemaphoreType.DMA,
        pltpu.VMEM_SHARED((NPAD,), jnp.float32),
        pltpu.VMEM_SHARED((NPADR, DH), jnp.float32),
    ],
  )
  return ka_call, kb_call


def kernel(h, edge_index, h_attn_q, edge_attr, We, be, Wa, ba):
    srcf = edge_index[0]
    dstf = edge_index[1]
    src3 = srcf.reshape(NS, NCHUNK2, K)
    dst3 = dstf.reshape(NS, NCHUNK2, K)
    hsplit = h.reshape(N, NC, DH).transpose(1, 0, 2)
    ea0 = edge_attr[:, 0]
    ea1 = edge_attr[:, 1]
    w1 = Wa[:D, 0]
    w2 = Wa[D:, 0]
    ew = We @ w2
    c0 = be @ w2 + ba[0]
    par = jnp.zeros((16,), jnp.float32).at[0].set(ew[0]).at[1].set(ew[1]) \
        .at[2].set(c0)

    qs2, hs2 = _pre_call(h, h_attn_q, w1.reshape(1, D), w2.reshape(1, D))
    qs = qs2.reshape(N)
    hs = hs2.reshape(N)

    ka_call, kb_call = _sc_calls()
    dst3a = dstf.reshape(NW, NCHUNK, K)
    ex, den, smax = ka_call(srcf, dst3a, ea0, ea1, qs, hs, par)
    outp = kb_call(src3, dst3, ex, den, smax, hsplit)
    return _post_call(outp[0], outp[1])


# speedup vs baseline: 14.8405x; 1.0200x over previous
import functools

import jax
import jax.numpy as jnp
from jax import lax
from jax.experimental import pallas as pl
from jax.experimental.pallas import tpu as pltpu
from jax.experimental.pallas import tpu_sc as plsc

N = 10000
E = 320000
D = 128

NC = 2
NS = 16
NW = NC * NS
EPT = E // NW
K = 80
NCHUNK = EPT // K
NPAD = 10240
NSLICE = NPAD // NS
NPADR = 10240
ORPT = NPADR // NS
OZROWS = 64


def _pre_body(h_ref, hq_ref, w1_ref, w2_ref, qs_ref, hs_ref):
    qs_ref[...] = jnp.sum(hq_ref[...] * w1_ref[...], axis=1, keepdims=True)
    hs_ref[...] = jnp.sum(h_ref[...] * w2_ref[...], axis=1, keepdims=True)


def _post_body(a_ref, b_ref, o_ref):
    o_ref[...] = jnp.concatenate(
        [a_ref[pl.ds(0, N), :], b_ref[pl.ds(0, N), :]], axis=1)


def _sca_body(srcf_hbm, dst3_hbm, ea0_hbm, ea1_hbm, qs_hbm, hs_hbm,
              par_hbm,
              ex_hbm, den_hbm, smax_hbm,
              srcf, dstc, ea0t, ea1t, qst, hst, part, at_, ext, mxt,
              mxall, zb, asem, dsem, sden, smaxs):
    c = lax.axis_index("c")
    s = lax.axis_index("s")
    wid = c * NS + s
    ebase = wid * EPT

    pltpu.async_copy(srcf_hbm.at[pl.ds(ebase, EPT)], srcf, asem)
    pltpu.async_copy(dst3_hbm.at[wid], dstc, asem)
    pltpu.async_copy(ea0_hbm.at[pl.ds(ebase, EPT)], ea0t, asem)
    pltpu.async_copy(ea1_hbm.at[pl.ds(ebase, EPT)], ea1t, asem)
    pltpu.async_copy(qs_hbm, qst, asem)
    pltpu.async_copy(hs_hbm, hst, asem)
    pltpu.async_copy(par_hbm, part, asem)
    pltpu.make_async_copy(srcf_hbm.at[pl.ds(ebase, EPT)], srcf, asem).wait()
    pltpu.make_async_copy(dst3_hbm.at[wid], dstc, asem).wait()
    pltpu.make_async_copy(ea0_hbm.at[pl.ds(ebase, EPT)], ea0t, asem).wait()
    pltpu.make_async_copy(ea1_hbm.at[pl.ds(ebase, EPT)], ea1t, asem).wait()
    pltpu.make_async_copy(qs_hbm, qst, asem).wait()
    pltpu.make_async_copy(hs_hbm, hst, asem).wait()
    pltpu.make_async_copy(par_hbm, part, asem).wait()
    pv = part[...]
    ew0 = pv[0]
    ew1 = pv[1]
    c0 = pv[2]

    def _zb_step(i, _):
        zb[pl.ds(i * 16, 16)] = jnp.zeros((16,), jnp.float32)
        return 0
    lax.fori_loop(0, NSLICE // 16, _zb_step, 0)
    pltpu.sync_copy(zb, sden.at[pl.ds(s * NSLICE, NSLICE)])

    def _step_a(j, mx):
        for k in range(K // 16):
            sl = pl.ds(j * K + k * 16, 16)
            a16 = (plsc.load_gather(qst, [dstc[j, pl.ds(k * 16, 16)]])
                   + plsc.load_gather(hst, [srcf[sl]])
                   + ea0t[sl] * ew0 + ea1t[sl] * ew1 + c0)
            at_[sl] = a16
            mx = jnp.maximum(mx, a16)
        return mx
    mx = lax.fori_loop(0, NCHUNK, _step_a,
                       jnp.full((16,), -jnp.inf, jnp.float32))
    mxt[...] = mx
    pltpu.sync_copy(mxt, smaxs.at[s])

    plsc.subcore_barrier()

    pltpu.sync_copy(smaxs, mxall)
    mm = mxall[0]
    for s2 in range(1, NS):
        mm = jnp.maximum(mm, mxall[s2])
    m_s = jnp.max(mm)

    def _chunk_b(j, _):
        @pl.when(j >= 2)
        def _():
            pltpu.make_async_copy(ext.at[pl.ds(0, K)], sden.at[dstc.at[0]],
                                  dsem).wait()
        for k in range(K // 16):
            sl = pl.ds(j * K + k * 16, 16)
            ext[sl] = jnp.exp(at_[sl] - m_s)
        pltpu.async_copy(ext.at[pl.ds(j * K, K)], sden.at[dstc.at[j]], dsem,
                         add=True)
        return 0
    lax.fori_loop(0, NCHUNK, _chunk_b, 0)
    pltpu.make_async_copy(ext.at[pl.ds(0, K)], sden.at[dstc.at[0]],
                          dsem).wait()
    pltpu.make_async_copy(ext.at[pl.ds(0, K)], sden.at[dstc.at[0]],
                          dsem).wait()
    pltpu.sync_copy(ext, ex_hbm.at[pl.ds(ebase, EPT)])

    @pl.when(s == 0)
    def _():
        mxt[...] = jnp.full((16,), m_s, jnp.float32)
        pltpu.sync_copy(mxt, smax_hbm.at[pl.ds(c * 16, 16)])

    plsc.subcore_barrier()
    pltpu.sync_copy(sden.at[pl.ds(s * NSLICE, NSLICE)],
                    den_hbm.at[pl.ds(c * NPAD + s * NSLICE, NSLICE)])


DH = D // NC
EPT2 = E // NS
NCHUNK2 = EPT2 // K


def _scb_body(src3_hbm, dst3_hbm, ex_hbm, den_hbm, smax_hbm, hsplit_hbm,
              outp_hbm,
              srcc, dstc, exc, dent, d0s, d1s, smaxt, ab, rows0, rows1, zb,
              gsem0, gsem1, ssem0, ssem1, sdc, sout):
    c = lax.axis_index("c")
    s = lax.axis_index("s")
    ebase = s * EPT2

    pltpu.async_copy(src3_hbm.at[s], srcc, gsem0)
    pltpu.async_copy(dst3_hbm.at[s], dstc, gsem0)
    pltpu.async_copy(ex_hbm.at[pl.ds(ebase, EPT2)], exc, gsem0)
    pltpu.async_copy(den_hbm.at[pl.ds(s * NSLICE, NSLICE)], d0s, gsem0)
    pltpu.async_copy(den_hbm.at[pl.ds(NPAD + s * NSLICE, NSLICE)], d1s, gsem0)
    pltpu.async_copy(smax_hbm, smaxt, gsem0)
    pltpu.make_async_copy(src3_hbm.at[s], srcc, gsem0).wait()
    pltpu.make_async_copy(dst3_hbm.at[s], dstc, gsem0).wait()
    pltpu.make_async_copy(ex_hbm.at[pl.ds(ebase, EPT2)], exc, gsem0).wait()
    pltpu.make_async_copy(den_hbm.at[pl.ds(s * NSLICE, NSLICE)], d0s,
                          gsem0).wait()
    pltpu.make_async_copy(den_hbm.at[pl.ds(NPAD + s * NSLICE, NSLICE)], d1s,
                          gsem0).wait()
    pltpu.make_async_copy(smax_hbm, smaxt, gsem0).wait()

    m0 = smaxt[pl.ds(0, 16)]
    m1 = smaxt[pl.ds(16, 16)]
    mg = jnp.maximum(m0, m1)
    s0 = jnp.exp(m0 - mg)
    s1 = jnp.exp(m1 - mg)
    myv = jnp.where(s < NS // 2, s0, s1)

    def _den_step(i, _):
        sl = pl.ds(i * 16, 16)
        d0s[sl] = d0s[sl] * s0 + d1s[sl] * s1
        return 0
    lax.fori_loop(0, NSLICE // 16, _den_step, 0)
    pltpu.sync_copy(d0s, sdc.at[pl.ds(s * NSLICE, NSLICE)])

    def _zb_step(i, _):
        r = i // (DH // 16)
        u = i % (DH // 16)
        zb[r, pl.ds(u * 16, 16)] = jnp.zeros((16,), jnp.float32)
        return 0
    lax.fori_loop(0, OZROWS * (DH // 16), _zb_step, 0)
    for t in range(ORPT // OZROWS):
        pltpu.sync_copy(zb, sout.at[pl.ds(s * ORPT + t * OZROWS, OZROWS)])

    plsc.subcore_barrier()
    pltpu.sync_copy(sdc, dent)

    def _inv_step(i, _):
        sl = pl.ds(i * 16, 16)
        dent[sl] = 1.0 / (dent[sl] + 1e-16)
        return 0
    lax.fori_loop(0, NPAD // 16, _inv_step, 0)

    rowsb = (rows0, rows1)
    gsems = (gsem0, gsem1)
    ssems = (ssem0, ssem1)

    pltpu.async_copy(hsplit_hbm.at[c].at[srcc.at[0]], rows0, gsem0)

    def _pair(pp, _):
        for b in range(2):
            j = pp * 2 + b
            bn = 1 - b
            jn = j + 1

            @pl.when(j >= 1)
            def _():
                pltpu.make_async_copy(rowsb[bn], sout.at[dstc.at[j]],
                                      ssems[bn]).wait()

            @pl.when(jn < NCHUNK2)
            def _():
                pltpu.async_copy(hsplit_hbm.at[c].at[srcc.at[jn]],
                                 rowsb[bn], gsems[bn])

            pltpu.make_async_copy(hsplit_hbm.at[c].at[srcc.at[j]],
                                  rowsb[b], gsems[b]).wait()
            for k in range(K // 16):
                sl = pl.ds(k * 16, 16)
                dstv = dstc[j, sl]
                denv = plsc.load_gather(dent, [dstv])
                ab[sl] = exc[pl.ds(j * K + k * 16, 16)] * myv * denv

            def _rowgrp(g, _2):
                al16 = ab[pl.ds(g * 16, 16)]
                for i in range(16):
                    r = g * 16 + i
                    al = al16[i]
                    for u in range(DH // 16):
                        su = pl.ds(u * 16, 16)
                        rowsb[b][r, su] = rowsb[b][r, su] * al
                return 0
            lax.fori_loop(0, K // 16, _rowgrp, 0)
            pltpu.async_copy(rowsb[b], sout.at[dstc.at[j]], ssems[b],
                             add=True)
        return 0
    lax.fori_loop(0, NCHUNK2 // 2, _pair, 0)
    pltpu.make_async_copy(rows1, sout.at[dstc.at[0]], ssem1).wait()

    plsc.subcore_barrier()
    pltpu.sync_copy(sout.at[pl.ds(s * ORPT, ORPT)],
                    outp_hbm.at[c, pl.ds(s * ORPT, ORPT)])


_pre_call = pl.pallas_call(
    _pre_body,
    out_shape=(jax.ShapeDtypeStruct((N, 1), jnp.float32),
               jax.ShapeDtypeStruct((N, 1), jnp.float32)),
)

_post_call = pl.pallas_call(
    _post_body,
    out_shape=jax.ShapeDtypeStruct((N, D), jnp.float32),
)


@functools.cache
def _sc_calls():
  mesh = plsc.VectorSubcoreMesh(core_axis_name="c", subcore_axis_name="s",
                                num_cores=NC, num_subcores=NS)
  cp = pltpu.CompilerParams(needs_layout_passes=False,
                            use_tc_tiling_on_sc=False)
  ka_call = pl.kernel(
    _sca_body,
    compiler_params=cp,
    out_type=(jax.ShapeDtypeStruct((E,), jnp.float32),
              jax.ShapeDtypeStruct((NC * NPAD,), jnp.float32),
              jax.ShapeDtypeStruct((NC * 16,), jnp.float32)),
    mesh=mesh,
    scratch_types=[
        pltpu.VMEM((EPT,), jnp.int32),
        pltpu.VMEM((NCHUNK, K), jnp.int32),
        pltpu.VMEM((EPT,), jnp.float32),
        pltpu.VMEM((EPT,), jnp.float32),
        pltpu.VMEM((N,), jnp.float32),
        pltpu.VMEM((N,), jnp.float32),
        pltpu.VMEM((16,), jnp.float32),
        pltpu.VMEM((EPT,), jnp.float32),
        pltpu.VMEM((EPT,), jnp.float32),
        pltpu.VMEM((16,), jnp.float32),
        pltpu.VMEM((NS, 16), jnp.float32),
        pltpu.VMEM((NSLICE,), jnp.float32),
        pltpu.SemaphoreType.DMA,
        pltpu.SemaphoreType.DMA,
        pltpu.VMEM_SHARED((NPAD,), jnp.float32),
        pltpu.VMEM_SHARED((NS, 16), jnp.float32),
    ],
  )
  kb_call = pl.kernel(
    _scb_body,
    compiler_params=cp,
    out_type=jax.ShapeDtypeStruct((NC, NPADR, DH), jnp.float32),
    mesh=mesh,
    scratch_types=[
        pltpu.VMEM((NCHUNK2, K), jnp.int32),
        pltpu.VMEM((NCHUNK2, K), jnp.int32),
        pltpu.VMEM((EPT2,), jnp.float32),
        pltpu.VMEM((NPAD,), jnp.float32),
        pltpu.VMEM((NSLICE,), jnp.float32),
        pltpu.VMEM((NSLICE,), jnp.float32),
        pltpu.VMEM((NC * 16,), jnp.float32),
        pltpu.VMEM((K,), jnp.float32),
        pltpu.VMEM((K, DH), jnp.float32),
        pltpu.VMEM((K, DH), jnp.float32),
        pltpu.VMEM((OZROWS, DH), jnp.float32),
        pltpu.SemaphoreType.DMA,
        pltpu.SemaphoreType.DMA,
        pltpu.SemaphoreType.DMA,
        pltpu.SemaphoreType.DMA,
        pltpu.VMEM_SHARED((NPAD,), jnp.float32),
        pltpu.VMEM_SHARED((NPADR, DH), jnp.float32),
    ],
  )
  return ka_call, kb_call


def kernel(h, edge_index, h_attn_q, edge_attr, We, be, Wa, ba):
    srcf = edge_index[0]
    dstf = edge_index[1]
    src3 = srcf.reshape(NS, NCHUNK2, K)
    dst3 = dstf.reshape(NS, NCHUNK2, K)
    hsplit = h.reshape(N, NC, DH).transpose(1, 0, 2)
    ea0 = edge_attr[:, 0]
    ea1 = edge_attr[:, 1]
    w1 = Wa[:D, 0]
    w2 = Wa[D:, 0]
    ew = We @ w2
    c0 = be @ w2 + ba[0]
    par = jnp.zeros((16,), jnp.float32).at[0].set(ew[0]).at[1].set(ew[1]) \
        .at[2].set(c0)

    qs2, hs2 = _pre_call(h, h_attn_q, w1.reshape(1, D), w2.reshape(1, D))
    qs = qs2.reshape(N)
    hs = hs2.reshape(N)

    ka_call, kb_call = _sc_calls()
    dst3a = dstf.reshape(NW, NCHUNK, K)
    ex, den, smax = ka_call(srcf, dst3a, ea0, ea1, qs, hs, par)
    outp = kb_call(src3, dst3, ex, den, smax, hsplit)
    return _post_call(outp[0], outp[1])
